# baseline (speedup 1.0000x reference)
import functools

import jax
import jax.numpy as jnp
from jax import lax
from jax.experimental import pallas as pl
from jax.experimental.pallas import tpu as pltpu
from jax.experimental.pallas import tpu_sc as plsc

N = 10000
E = 320000
DIN = 128
H1 = 8
C1 = 8
F1 = H1 * C1
COUT = 40
CPAD = 48

NPAD = 10240
BLK = 512
EB = 3200

EPC = E // 2
NBLK = EPC // EB


def _zero_ref(ref, nwords):
    zf = jnp.zeros((16,), jnp.float32)

    @plsc.parallel_loop(0, nwords // 16, unroll=8)
    def _(i):
        ref[pl.ds(i * 16, 16)] = zf


def _edge_pass(src_hbm, dst_hbm, hs, accs, asrc, adst, dacc, sbufs, dbufs,
               sems, base, nchan):
    def start_pair(kb, b):
        off = base + kb * EB
        pltpu.async_copy(src_hbm.at[pl.ds(off, EB)], sbufs[b], sems[b])
        pltpu.async_copy(dst_hbm.at[pl.ds(off, EB)], dbufs[b], sems[b])

    def wait_pair(b):
        pltpu.make_async_copy(src_hbm.at[pl.ds(0, EB)], sbufs[b], sems[b]).wait()
        pltpu.make_async_copy(dst_hbm.at[pl.ds(0, EB)], dbufs[b], sems[b]).wait()

    start_pair(0, 0)

    def outer(k2, carry):
        for b in range(2):
            kb = 2 * k2 + b

            @pl.when(kb + 1 < NBLK)
            def _():
                start_pair(kb + 1, 1 - b)

            wait_pair(b)
            sbuf = sbufs[b]
            dbuf = dbufs[b]

            @plsc.parallel_loop(0, EB // 16, unroll=2)
            def _(g):
                sv = sbuf[pl.ds(g * 16, 16)]
                dv = dbuf[pl.ds(g * 16, 16)]
                a_s = plsc.load_gather(asrc, [sv])
                a_d = plsc.load_gather(adst, [dv])
                e = a_s + a_d
                e = jnp.where(e >= 0.0, e, 0.2 * e)
                ex = jnp.exp(e)
                plsc.addupdate_scatter(dacc, [dv], ex)
                for cl in range(nchan):
                    hv = plsc.load_gather(hs[cl], [sv])
                    plsc.addupdate_scatter(accs[cl], [dv], ex * hv)
        return carry
    lax.fori_loop(0, NBLK // 2, outer, 0)


def _tc1_body(x_ref, w1_ref, aa_ref, h1t_ref, a1t_ref):
    xb = x_ref[...]
    h1t = lax.dot_general(w1_ref[...], xb, (((0,), (1,)), ((), ())),
                          preferred_element_type=jnp.float32)
    h1t_ref[...] = h1t
    a1t_ref[...] = jnp.dot(aa_ref[...], h1t, preferred_element_type=jnp.float32)


def _tc1(xp, W1, AA):
    return pl.pallas_call(
        _tc1_body,
        grid=(NPAD // BLK,),
        in_specs=[
            pl.BlockSpec((BLK, DIN), lambda i: (i, 0)),
            pl.BlockSpec((DIN, F1), lambda i: (0, 0)),
            pl.BlockSpec((2 * H1, F1), lambda i: (0, 0)),
        ],
        out_specs=[
            pl.BlockSpec((F1, BLK), lambda i: (0, i)),
            pl.BlockSpec((2 * H1, BLK), lambda i: (0, i)),
        ],
        out_shape=[
            jax.ShapeDtypeStruct((F1, NPAD), jnp.float32),
            jax.ShapeDtypeStruct((2 * H1, NPAD), jnp.float32),
        ],
    )(xp, W1, AA)


def _sc1_body(src_hbm, dst_hbm, h1t_hbm, a1t_hbm, num_hbm, den_hbm,
              h10, h11, h12, h13, acc0, acc1, acc2, acc3, asrc, adst, dacc,
              sbuf0, sbuf1, dbuf0, dbuf1, sem0, sem1):
    c = lax.axis_index("c")
    s = lax.axis_index("s")
    head = s // 2
    hs = (h10, h11, h12, h13)
    accs = (acc0, acc1, acc2, acc3)

    for cl in range(4):
        pltpu.sync_copy(h1t_hbm.at[pl.ds((4 * s + cl) * NPAD, NPAD)], hs[cl])
    pltpu.sync_copy(a1t_hbm.at[pl.ds(head * NPAD, NPAD)], asrc)
    pltpu.sync_copy(a1t_hbm.at[pl.ds((H1 + head) * NPAD, NPAD)], adst)

    for cl in range(4):
        _zero_ref(accs[cl], NPAD)
    _zero_ref(dacc, NPAD)

    _edge_pass(src_hbm, dst_hbm, hs, accs, asrc, adst, dacc,
               (sbuf0, sbuf1), (dbuf0, dbuf1), (sem0, sem1),
               c * EPC, 4)

    for cl in range(4):
        pltpu.sync_copy(accs[cl],
                        num_hbm.at[pl.ds((F1 * c + 4 * s + cl) * NPAD, NPAD)])

    @pl.when(s % 2 == 0)
    def _():
        pltpu.sync_copy(dacc, den_hbm.at[pl.ds((H1 * c + head) * NPAD, NPAD)])


def _sc1(src, dst, h1t_flat, a1t_flat):
    mesh = plsc.VectorSubcoreMesh(core_axis_name="c", subcore_axis_name="s")
    f = functools.partial(
        pl.kernel,
        mesh=mesh,
        compiler_params=pltpu.CompilerParams(needs_layout_passes=False),
        out_type=[
            jax.ShapeDtypeStruct((2 * F1 * NPAD,), jnp.float32),
            jax.ShapeDtypeStruct((2 * H1 * NPAD,), jnp.float32),
        ],
        scratch_types=(
            [pltpu.VMEM((NPAD,), jnp.float32)] * 11
            + [pltpu.VMEM((EB,), jnp.int32)] * 4
            + [pltpu.SemaphoreType.DMA] * 2
        ),
    )(_sc1_body)
    return f(src, dst, h1t_flat, a1t_flat)


def _tc2_body(num_ref, den_ref, w2_ref, a2_ref, b1_ref, r_ref, h2t_ref, a2t_ref):
    num = num_ref[0:F1, :] + num_ref[F1:2 * F1, :]
    den8 = den_ref[0:H1, :] + den_ref[H1:2 * H1, :]
    den64 = jnp.dot(r_ref[...], den8, preferred_element_type=jnp.float32)
    out1 = num / (den64 + 1e-16) + b1_ref[...]
    z = jnp.where(out1 > 0.0, out1, jnp.exp(out1) - 1.0)
    h2t = lax.dot_general(w2_ref[...], z, (((0,), (0,)), ((), ())),
                          preferred_element_type=jnp.float32)
    h2t_ref[0:COUT, :] = h2t
    h2t_ref[COUT:CPAD, :] = jnp.zeros((CPAD - COUT, BLK), jnp.float32)
    a2t_ref[...] = jnp.dot(a2_ref[...], h2t, preferred_element_type=jnp.float32)


def _tc2(num1, den1, W2, A2, b1c, R):
    return pl.pallas_call(
        _tc2_body,
        grid=(NPAD // BLK,),
        in_specs=[
            pl.BlockSpec((2 * F1, BLK), lambda i: (0, i)),
            pl.BlockSpec((2 * H1, BLK), lambda i: (0, i)),
            pl.BlockSpec((F1, COUT), lambda i: (0, 0)),
            pl.BlockSpec((2, COUT), lambda i: (0, 0)),
            pl.BlockSpec((F1, 1), lambda i: (0, 0)),
            pl.BlockSpec((F1, H1), lambda i: (0, 0)),
        ],
        out_specs=[
            pl.BlockSpec((CPAD, BLK), lambda i: (0, i)),
            pl.BlockSpec((2, BLK), lambda i: (0, i)),
        ],
        out_shape=[
            jax.ShapeDtypeStruct((CPAD, NPAD), jnp.float32),
            jax.ShapeDtypeStruct((2, NPAD), jnp.float32),
        ],
    )(num1, den1, W2, A2, b1c, R)


def _sc2_body(src_hbm, dst_hbm, h2t_hbm, a2t_hbm, num_hbm, den_hbm,
              h20, h21, h22, acc0, acc1, acc2, asrc, adst, dacc,
              sbuf0, sbuf1, dbuf0, dbuf1, sem0, sem1):
    c = lax.axis_index("c")
    s = lax.axis_index("s")
    hs = (h20, h21, h22)
    accs = (acc0, acc1, acc2)

    for cl in range(3):
        pltpu.sync_copy(h2t_hbm.at[pl.ds((3 * s + cl) * NPAD, NPAD)], hs[cl])
    pltpu.sync_copy(a2t_hbm.at[pl.ds(0, NPAD)], asrc)
    pltpu.sync_copy(a2t_hbm.at[pl.ds(NPAD, NPAD)], adst)

    for cl in range(3):
        _zero_ref(accs[cl], NPAD)
    _zero_ref(dacc, NPAD)

    _edge_pass(src_hbm, dst_hbm, hs, accs, asrc, adst, dacc,
               (sbuf0, sbuf1), (dbuf0, dbuf1), (sem0, sem1),
               c * EPC, 3)

    for cl in range(3):
        pltpu.sync_copy(accs[cl],
                        num_hbm.at[pl.ds((CPAD * c + 3 * s + cl) * NPAD, NPAD)])

    @pl.when(s == 0)
    def _():
        pltpu.sync_copy(dacc, den_hbm.at[pl.ds(c * NPAD, NPAD)])


def _sc2(src, dst, h2t_flat, a2t_flat):
    mesh = plsc.VectorSubcoreMesh(core_axis_name="c", subcore_axis_name="s")
    f = functools.partial(
        pl.kernel,
        mesh=mesh,
        compiler_params=pltpu.CompilerParams(needs_layout_passes=False),
        out_type=[
            jax.ShapeDtypeStruct((2 * CPAD * NPAD,), jnp.float32),
            jax.ShapeDtypeStruct((2 * NPAD,), jnp.float32),
        ],
        scratch_types=(
            [pltpu.VMEM((NPAD,), jnp.float32)] * 9
            + [pltpu.VMEM((EB,), jnp.int32)] * 4
            + [pltpu.SemaphoreType.DMA] * 2
        ),
    )(_sc2_body)
    return f(src, dst, h2t_flat, a2t_flat)


def _tc3_body(num_ref, den_ref, b2_ref, out_ref):
    num = num_ref[0:CPAD, :] + num_ref[CPAD:2 * CPAD, :]
    den = den_ref[0:1, :] + den_ref[1:2, :]
    val = num[0:COUT, :] / (den + 1e-16) + b2_ref[...]
    out_ref[...] = jax.nn.sigmoid(val)


def _tc3(num2, den2, b2c):
    return pl.pallas_call(
        _tc3_body,
        grid=(NPAD // BLK,),
        in_specs=[
            pl.BlockSpec((2 * CPAD, BLK), lambda i: (0, i)),
            pl.BlockSpec((2, BLK), lambda i: (0, i)),
            pl.BlockSpec((COUT, 1), lambda i: (0, 0)),
        ],
        out_specs=pl.BlockSpec((COUT, BLK), lambda i: (0, i)),
        out_shape=jax.ShapeDtypeStruct((COUT, NPAD), jnp.float32),
    )(num2, den2, b2c)


def kernel(x, edge_index, W1, a_src1, a_dst1, b1, W2, a_src2, a_dst2, b2):
    src = edge_index[0]
    dst = edge_index[1]

    xp = jnp.pad(x, ((0, NPAD - N), (0, 0)))

    rows = jnp.arange(H1)[:, None]
    cols = rows * C1 + jnp.arange(C1)[None, :]
    As = jnp.zeros((H1, F1), jnp.float32).at[
        jnp.broadcast_to(rows, (H1, C1)), cols].set(a_src1)
    Ad = jnp.zeros((H1, F1), jnp.float32).at[
        jnp.broadcast_to(rows, (H1, C1)), cols].set(a_dst1)
    AA = jnp.concatenate([As, Ad], axis=0)

    h1t, a1t = _tc1(xp, W1, AA)
    num1, den1 = _sc1(src, dst, h1t.reshape(-1), a1t.reshape(-1))

    R = jnp.zeros((F1, H1), jnp.float32).at[
        jnp.arange(F1), jnp.arange(F1) // C1].set(1.0)
    A2 = jnp.concatenate([a_src2, a_dst2], axis=0)
    h2t, a2t = _tc2(num1.reshape(2 * F1, NPAD), den1.reshape(2 * H1, NPAD),
                    W2, A2, b1.reshape(F1, 1), R)

    num2, den2 = _sc2(src, dst, h2t.reshape(-1), a2t.reshape(-1))
    outt = _tc3(num2.reshape(2 * CPAD, NPAD), den2.reshape(2, NPAD),
                b2.reshape(COUT, 1))
    return outt[:, :N].T


# speedup vs baseline: 1.0518x; 1.0518x over previous
import functools

import jax
import jax.numpy as jnp
from jax import lax
from jax.experimental import pallas as pl
from jax.experimental.pallas import tpu as pltpu
from jax.experimental.pallas import tpu_sc as plsc

N = 10000
E = 320000
DIN = 128
H1 = 8
C1 = 8
F1 = H1 * C1
COUT = 40
CPAD = 48

NPAD = 10240
BLK = 1024
EB = 3200

EPC = E // 2
NBLK = EPC // EB


def _zero_ref(ref, nwords):
    zf = jnp.zeros((16,), jnp.float32)

    @plsc.parallel_loop(0, nwords // 16, unroll=8)
    def _(i):
        ref[pl.ds(i * 16, 16)] = zf


def _edge_pass(src_hbm, dst_hbm, hs, accs, asrc, adst, dacc, sbufs, dbufs,
               sems, base, nchan):
    def start_pair(kb, b):
        off = base + kb * EB
        pltpu.async_copy(src_hbm.at[pl.ds(off, EB)], sbufs[b], sems[b])
        pltpu.async_copy(dst_hbm.at[pl.ds(off, EB)], dbufs[b], sems[b])

    def wait_pair(b):
        pltpu.make_async_copy(src_hbm.at[pl.ds(0, EB)], sbufs[b], sems[b]).wait()
        pltpu.make_async_copy(dst_hbm.at[pl.ds(0, EB)], dbufs[b], sems[b]).wait()

    start_pair(0, 0)

    def outer(k2, carry):
        for b in range(2):
            kb = 2 * k2 + b

            @pl.when(kb + 1 < NBLK)
            def _():
                start_pair(kb + 1, 1 - b)

            wait_pair(b)
            sbuf = sbufs[b]
            dbuf = dbufs[b]

            @plsc.parallel_loop(0, EB // 16, unroll=2)
            def _(g):
                sv = sbuf[pl.ds(g * 16, 16)]
                dv = dbuf[pl.ds(g * 16, 16)]
                a_s = plsc.load_gather(asrc, [sv])
                a_d = plsc.load_gather(adst, [dv])
                e = a_s + a_d
                e = jnp.where(e >= 0.0, e, 0.2 * e)
                ex = jnp.exp(e)
                plsc.addupdate_scatter(dacc, [dv], ex)
                for cl in range(nchan):
                    hv = plsc.load_gather(hs[cl], [sv])
                    plsc.addupdate_scatter(accs[cl], [dv], ex * hv)
        return carry
    lax.fori_loop(0, NBLK // 2, outer, 0)


def _tc1_body(x_ref, w1_ref, aa_ref, h1t_ref, a1t_ref):
    xb = x_ref[...]
    h1t = lax.dot_general(w1_ref[...], xb, (((0,), (1,)), ((), ())),
                          preferred_element_type=jnp.float32)
    h1t_ref[...] = h1t
    a1t_ref[...] = jnp.dot(aa_ref[...], h1t, preferred_element_type=jnp.float32)


def _tc1(xp, W1, AA):
    return pl.pallas_call(
        _tc1_body,
        grid=(NPAD // BLK,),
        in_specs=[
            pl.BlockSpec((BLK, DIN), lambda i: (i, 0)),
            pl.BlockSpec((DIN, F1), lambda i: (0, 0)),
            pl.BlockSpec((2 * H1, F1), lambda i: (0, 0)),
        ],
        out_specs=[
            pl.BlockSpec((F1, BLK), lambda i: (0, i)),
            pl.BlockSpec((2 * H1, BLK), lambda i: (0, i)),
        ],
        out_shape=[
            jax.ShapeDtypeStruct((F1, NPAD), jnp.float32),
            jax.ShapeDtypeStruct((2 * H1, NPAD), jnp.float32),
        ],
    )(xp, W1, AA)


def _sc1_body(src_hbm, dst_hbm, h1t_hbm, a1t_hbm, num_hbm, den_hbm,
              h10, h11, h12, h13, acc0, acc1, acc2, acc3, asrc, adst, dacc,
              sbuf0, sbuf1, dbuf0, dbuf1, sem0, sem1):
    c = lax.axis_index("c")
    s = lax.axis_index("s")
    head = s // 2
    hs = (h10, h11, h12, h13)
    accs = (acc0, acc1, acc2, acc3)

    for cl in range(4):
        pltpu.sync_copy(h1t_hbm.at[pl.ds((4 * s + cl) * NPAD, NPAD)], hs[cl])
    pltpu.sync_copy(a1t_hbm.at[pl.ds(head * NPAD, NPAD)], asrc)
    pltpu.sync_copy(a1t_hbm.at[pl.ds((H1 + head) * NPAD, NPAD)], adst)

    for cl in range(4):
        _zero_ref(accs[cl], NPAD)
    _zero_ref(dacc, NPAD)

    _edge_pass(src_hbm, dst_hbm, hs, accs, asrc, adst, dacc,
               (sbuf0, sbuf1), (dbuf0, dbuf1), (sem0, sem1),
               c * EPC, 4)

    for cl in range(4):
        pltpu.sync_copy(accs[cl],
                        num_hbm.at[pl.ds((F1 * c + 4 * s + cl) * NPAD, NPAD)])

    @pl.when(s % 2 == 0)
    def _():
        pltpu.sync_copy(dacc, den_hbm.at[pl.ds((H1 * c + head) * NPAD, NPAD)])


def _sc1(src, dst, h1t_flat, a1t_flat):
    mesh = plsc.VectorSubcoreMesh(core_axis_name="c", subcore_axis_name="s")
    f = functools.partial(
        pl.kernel,
        mesh=mesh,
        compiler_params=pltpu.CompilerParams(needs_layout_passes=False),
        out_type=[
            jax.ShapeDtypeStruct((2 * F1 * NPAD,), jnp.float32),
            jax.ShapeDtypeStruct((2 * H1 * NPAD,), jnp.float32),
        ],
        scratch_types=(
            [pltpu.VMEM((NPAD,), jnp.float32)] * 11
            + [pltpu.VMEM((EB,), jnp.int32)] * 4
            + [pltpu.SemaphoreType.DMA] * 2
        ),
    )(_sc1_body)
    return f(src, dst, h1t_flat, a1t_flat)


def _tc2_body(num_ref, den_ref, w2_ref, a2_ref, b1_ref, r_ref, h2t_ref, a2t_ref):
    num = num_ref[0:F1, :] + num_ref[F1:2 * F1, :]
    den8 = den_ref[0:H1, :] + den_ref[H1:2 * H1, :]
    den64 = jnp.dot(r_ref[...], den8, preferred_element_type=jnp.float32)
    out1 = num / (den64 + 1e-16) + b1_ref[...]
    z = jnp.where(out1 > 0.0, out1, jnp.exp(out1) - 1.0)
    h2t = lax.dot_general(w2_ref[...], z, (((0,), (0,)), ((), ())),
                          preferred_element_type=jnp.float32)
    h2t_ref[0:COUT, :] = h2t
    h2t_ref[COUT:CPAD, :] = jnp.zeros((CPAD - COUT, BLK), jnp.float32)
    a2t_ref[...] = jnp.dot(a2_ref[...], h2t, preferred_element_type=jnp.float32)


def _tc2(num1, den1, W2, A2, b1c, R):
    return pl.pallas_call(
        _tc2_body,
        grid=(NPAD // BLK,),
        in_specs=[
            pl.BlockSpec((2 * F1, BLK), lambda i: (0, i)),
            pl.BlockSpec((2 * H1, BLK), lambda i: (0, i)),
            pl.BlockSpec((F1, COUT), lambda i: (0, 0)),
            pl.BlockSpec((2, COUT), lambda i: (0, 0)),
            pl.BlockSpec((F1, 1), lambda i: (0, 0)),
            pl.BlockSpec((F1, H1), lambda i: (0, 0)),
        ],
        out_specs=[
            pl.BlockSpec((CPAD, BLK), lambda i: (0, i)),
            pl.BlockSpec((2, BLK), lambda i: (0, i)),
        ],
        out_shape=[
            jax.ShapeDtypeStruct((CPAD, NPAD), jnp.float32),
            jax.ShapeDtypeStruct((2, NPAD), jnp.float32),
        ],
    )(num1, den1, W2, A2, b1c, R)


def _sc2_body(src_hbm, dst_hbm, h2t_hbm, a2t_hbm, num_hbm, den_hbm,
              h20, h21, h22, acc0, acc1, acc2, asrc, adst, dacc,
              sbuf0, sbuf1, dbuf0, dbuf1, sem0, sem1):
    c = lax.axis_index("c")
    s = lax.axis_index("s")
    hs = (h20, h21, h22)
    accs = (acc0, acc1, acc2)

    for cl in range(3):
        pltpu.sync_copy(h2t_hbm.at[pl.ds((3 * s + cl) * NPAD, NPAD)], hs[cl])
    pltpu.sync_copy(a2t_hbm.at[pl.ds(0, NPAD)], asrc)
    pltpu.sync_copy(a2t_hbm.at[pl.ds(NPAD, NPAD)], adst)

    for cl in range(3):
        _zero_ref(accs[cl], NPAD)
    _zero_ref(dacc, NPAD)

    _edge_pass(src_hbm, dst_hbm, hs, accs, asrc, adst, dacc,
               (sbuf0, sbuf1), (dbuf0, dbuf1), (sem0, sem1),
               c * EPC, 3)

    for cl in range(3):
        pltpu.sync_copy(accs[cl],
                        num_hbm.at[pl.ds((CPAD * c + 3 * s + cl) * NPAD, NPAD)])

    @pl.when(s == 0)
    def _():
        pltpu.sync_copy(dacc, den_hbm.at[pl.ds(c * NPAD, NPAD)])


def _sc2(src, dst, h2t_flat, a2t_flat):
    mesh = plsc.VectorSubcoreMesh(core_axis_name="c", subcore_axis_name="s")
    f = functools.partial(
        pl.kernel,
        mesh=mesh,
        compiler_params=pltpu.CompilerParams(needs_layout_passes=False),
        out_type=[
            jax.ShapeDtypeStruct((2 * CPAD * NPAD,), jnp.float32),
            jax.ShapeDtypeStruct((2 * NPAD,), jnp.float32),
        ],
        scratch_types=(
            [pltpu.VMEM((NPAD,), jnp.float32)] * 9
            + [pltpu.VMEM((EB,), jnp.int32)] * 4
            + [pltpu.SemaphoreType.DMA] * 2
        ),
    )(_sc2_body)
    return f(src, dst, h2t_flat, a2t_flat)


def _tc3_body(num_ref, den_ref, b2_ref, out_ref):
    num = num_ref[0:CPAD, :] + num_ref[CPAD:2 * CPAD, :]
    den = den_ref[0:1, :] + den_ref[1:2, :]
    val = num[0:COUT, :] / (den + 1e-16) + b2_ref[...]
    out_ref[...] = jax.nn.sigmoid(val)


def _tc3(num2, den2, b2c):
    return pl.pallas_call(
        _tc3_body,
        grid=(NPAD // BLK,),
        in_specs=[
            pl.BlockSpec((2 * CPAD, BLK), lambda i: (0, i)),
            pl.BlockSpec((2, BLK), lambda i: (0, i)),
            pl.BlockSpec((COUT, 1), lambda i: (0, 0)),
        ],
        out_specs=pl.BlockSpec((COUT, BLK), lambda i: (0, i)),
        out_shape=jax.ShapeDtypeStruct((COUT, NPAD), jnp.float32),
    )(num2, den2, b2c)


def kernel(x, edge_index, W1, a_src1, a_dst1, b1, W2, a_src2, a_dst2, b2):
    src = edge_index[0]
    dst = edge_index[1]

    xp = jnp.pad(x, ((0, NPAD - N), (0, 0)))

    rows = jnp.arange(H1)[:, None]
    cols = rows * C1 + jnp.arange(C1)[None, :]
    As = jnp.zeros((H1, F1), jnp.float32).at[
        jnp.broadcast_to(rows, (H1, C1)), cols].set(a_src1)
    Ad = jnp.zeros((H1, F1), jnp.float32).at[
        jnp.broadcast_to(rows, (H1, C1)), cols].set(a_dst1)
    AA = jnp.concatenate([As, Ad], axis=0)

    h1t, a1t = _tc1(xp, W1, AA)
    num1, den1 = _sc1(src, dst, h1t.reshape(-1), a1t.reshape(-1))

    R = jnp.zeros((F1, H1), jnp.float32).at[
        jnp.arange(F1), jnp.arange(F1) // C1].set(1.0)
    A2 = jnp.concatenate([a_src2, a_dst2], axis=0)
    h2t, a2t = _tc2(num1.reshape(2 * F1, NPAD), den1.reshape(2 * H1, NPAD),
                    W2, A2, b1.reshape(F1, 1), R)

    num2, den2 = _sc2(src, dst, h2t.reshape(-1), a2t.reshape(-1))
    outt = _tc3(num2.reshape(2 * CPAD, NPAD), den2.reshape(2, NPAD),
                b2.reshape(COUT, 1))
    return outt[:, :N].T
